# baseline (speedup 1.0000x reference)
import functools

import jax
import jax.numpy as jnp
from jax import lax
from jax.experimental import pallas as pl
from jax.experimental.pallas import tpu as pltpu
from jax.experimental.pallas import tpu_sc as plsc

N_NODES = 10000
D_FEAT = 128
N_EDGES = 320000

_NC = 2
_NS = 16
_NW = _NC * _NS
_PER_W = N_EDGES // _NW
_C = 128
_NCHUNK = 79
_PAD_W = _NCHUNK * _C

_LANES = 16
_KVEC = D_FEAT // _LANES
_NBUF = 4


def _sc_kernel(xd, xs, idx0, idx1, out,
               i0all, i1all, r0s, r1s, ov, pv, s0s, s1s):
    wid = lax.axis_index("s") * _NC + lax.axis_index("c")
    pltpu.sync_copy(idx0.at[wid], i0all)
    pltpu.sync_copy(idx1.at[wid], i1all)

    lane = lax.iota(jnp.int32, _LANES)

    def issue(g, r0, r1, s0, s1):
        pltpu.async_copy(xd.at[i0all.at[g]], r0, s0)
        pltpu.async_copy(xs.at[i1all.at[g]], r1, s1)

    def wait(g, r0, r1, s0, s1):
        pltpu.make_async_copy(xd.at[i0all.at[g]], r0, s0).wait()
        pltpu.make_async_copy(xs.at[i1all.at[g]], r1, s1).wait()

    def compute(g, r0, r1):
        def p1_body(gr, gcarry):
            for t in range(2):
                base = gr * _LANES + t * 8
                ejs = [base + j for j in range(8)]
                for k in range(_KVEC // 2):
                    aa = [plsc.bitcast(r0[e, pl.ds(k * _LANES, _LANES)],
                                       jnp.bfloat16) for e in ejs]
                    bb = [plsc.bitcast(r1[e, pl.ds(k * _LANES, _LANES)],
                                       jnp.bfloat16) for e in ejs]
                    pp = [a * b for a, b in zip(aa, bb)]
                    if k == 0:
                        ss = pp
                    elif k == 1:
                        ps = pp
                    elif k == 2:
                        ss = [s + p for s, p in zip(ss, pp)]
                    else:
                        ps = [s + p for s, p in zip(ps, pp)]
                ss = [s + p for s, p in zip(ss, ps)]
                ups = [plsc.unpack(s, format=plsc.PackFormat.INTERLEAVED,
                                   preferred_element_type=jnp.float32)
                       for s in ss]
                res = [u0 + u1 for u0, u1 in ups]
                for j in range(8):
                    pv[base + j, pl.ds(0, _LANES)] = res[j]
            return gcarry

        lax.fori_loop(0, _C // _LANES, p1_body, 0)

        def p2_body(gr, gcarry):
            rows = gr * _LANES + lane
            cols = [plsc.load_gather(pv, [rows, jnp.full((_LANES,), c, jnp.int32)])
                    for c in range(_LANES)]
            for step in (8, 4, 2, 1):
                cols = [cols[2 * t] + cols[2 * t + 1] for t in range(step)]
            ov[pl.ds(g * _C + gr * _LANES, _LANES)] = cols[0]
            return gcarry

        lax.fori_loop(0, _C // _LANES, p2_body, 0)

    for b in range(_NBUF):
        issue(b, r0s[b], r1s[b], s0s[b], s1s[b])

    def ring_body(i, carry):
        for b in range(_NBUF):
            g = _NBUF * i + b
            wait(g, r0s[b], r1s[b], s0s[b], s1s[b])
            compute(g, r0s[b], r1s[b])

            @pl.when(g + _NBUF < _NCHUNK)
            def _():
                issue(g + _NBUF, r0s[b], r1s[b], s0s[b], s1s[b])

        return carry

    full = _NCHUNK // _NBUF
    lax.fori_loop(0, full, ring_body, 0)
    for g in range(full * _NBUF, _NCHUNK):
        b = g % _NBUF
        wait(g, r0s[b], r1s[b], s0s[b], s1s[b])
        compute(g, r0s[b], r1s[b])

    pltpu.sync_copy(ov, out.at[wid])


@jax.jit
def _run(x_disease, x_snorna, idx0, idx1):
    mesh = plsc.VectorSubcoreMesh(core_axis_name="c", subcore_axis_name="s")
    f = functools.partial(
        pl.kernel,
        mesh=mesh,
        out_type=jax.ShapeDtypeStruct((_NW, _PAD_W), jnp.float32),
        scratch_types=[
            pltpu.VMEM((_NCHUNK, _C), jnp.int32),
            pltpu.VMEM((_NCHUNK, _C), jnp.int32),
            [pltpu.VMEM((_C, D_FEAT // 2), jnp.int32)] * _NBUF,
            [pltpu.VMEM((_C, D_FEAT // 2), jnp.int32)] * _NBUF,
            pltpu.VMEM((_PAD_W,), jnp.float32),
            pltpu.VMEM((_C, 40), jnp.float32),
            [pltpu.SemaphoreType.DMA] * _NBUF,
            [pltpu.SemaphoreType.DMA] * _NBUF,
        ],
        compiler_params=pltpu.CompilerParams(needs_layout_passes=False,
                                             use_tc_tiling_on_sc=False),
    )(_sc_kernel)
    return f(x_disease, x_snorna, idx0, idx1)


def kernel(x_disease, x_snorna, edge_label_index):
    pad = ((0, 0), (0, _PAD_W - _PER_W))
    idx0 = jnp.pad(edge_label_index[0].reshape(_NW, _PER_W),
                   pad).reshape(_NW, _NCHUNK, _C)
    idx1 = jnp.pad(edge_label_index[1].reshape(_NW, _PER_W),
                   pad).reshape(_NW, _NCHUNK, _C)
    xd = lax.bitcast_convert_type(
        x_disease.astype(jnp.bfloat16).reshape(N_NODES, D_FEAT // 2, 2),
        jnp.int32)
    xs = lax.bitcast_convert_type(
        x_snorna.astype(jnp.bfloat16).reshape(N_NODES, D_FEAT // 2, 2),
        jnp.int32)
    return _run(xd, xs, idx0, idx1)[:, :_PER_W].reshape(N_EDGES)


# speedup vs baseline: 1.0773x; 1.0773x over previous
import functools

import jax
import jax.numpy as jnp
from jax import lax
from jax.experimental import pallas as pl
from jax.experimental.pallas import tpu as pltpu
from jax.experimental.pallas import tpu_sc as plsc

N_NODES = 10000
D_FEAT = 128
N_EDGES = 320000

_NC = 2
_NS = 16
_NW = _NC * _NS
_PER_W = N_EDGES // _NW
_C = 112
_NCHUNK = 90
_PAD_W = _NCHUNK * _C

_LANES = 16
_KVEC = D_FEAT // _LANES
_NBUF = 4


def _sc_kernel(xd, xs, idx0, idx1, out,
               i0all, i1all, r0s, r1s, ov, pv, s0s, s1s):
    wid = lax.axis_index("s") * _NC + lax.axis_index("c")
    pltpu.sync_copy(idx0.at[wid], i0all)
    pltpu.sync_copy(idx1.at[wid], i1all)

    lane = lax.iota(jnp.int32, _LANES)

    def issue(g, r0, r1, s0, s1):
        pltpu.async_copy(xd.at[i0all.at[g]], r0, s0)
        pltpu.async_copy(xs.at[i1all.at[g]], r1, s1)

    def wait(g, r0, r1, s0, s1):
        pltpu.make_async_copy(xd.at[i0all.at[g]], r0, s0).wait()
        pltpu.make_async_copy(xs.at[i1all.at[g]], r1, s1).wait()

    def compute(g, r0, r1):
        def p1_body(gr, gcarry):
            for t in range(2):
                base = gr * _LANES + t * 8
                ejs = [base + j for j in range(8)]
                for k in range(_KVEC // 2):
                    aa = [plsc.bitcast(r0[e, pl.ds(k * _LANES, _LANES)],
                                       jnp.bfloat16) for e in ejs]
                    bb = [plsc.bitcast(r1[e, pl.ds(k * _LANES, _LANES)],
                                       jnp.bfloat16) for e in ejs]
                    pp = [a * b for a, b in zip(aa, bb)]
                    if k == 0:
                        ss = pp
                    elif k == 1:
                        ps = pp
                    elif k == 2:
                        ss = [s + p for s, p in zip(ss, pp)]
                    else:
                        ps = [s + p for s, p in zip(ps, pp)]
                ss = [s + p for s, p in zip(ss, ps)]
                ups = [plsc.unpack(s, format=plsc.PackFormat.INTERLEAVED,
                                   preferred_element_type=jnp.float32)
                       for s in ss]
                res = [u0 + u1 for u0, u1 in ups]
                for j in range(8):
                    pv[base + j, pl.ds(0, _LANES)] = res[j]
            return gcarry

        lax.fori_loop(0, _C // _LANES, p1_body, 0)

        def p2_body(gr, gcarry):
            rows = gr * _LANES + lane
            cols = [plsc.load_gather(pv, [rows, jnp.full((_LANES,), c, jnp.int32)])
                    for c in range(_LANES)]
            for step in (8, 4, 2, 1):
                cols = [cols[2 * t] + cols[2 * t + 1] for t in range(step)]
            ov[pl.ds(g * _C + gr * _LANES, _LANES)] = cols[0]
            return gcarry

        lax.fori_loop(0, _C // _LANES, p2_body, 0)

    for b in range(_NBUF):
        issue(b, r0s[b], r1s[b], s0s[b], s1s[b])

    def ring_body(i, carry):
        for b in range(_NBUF):
            g = _NBUF * i + b
            wait(g, r0s[b], r1s[b], s0s[b], s1s[b])
            compute(g, r0s[b], r1s[b])

            @pl.when(g + _NBUF < _NCHUNK)
            def _():
                issue(g + _NBUF, r0s[b], r1s[b], s0s[b], s1s[b])

        return carry

    full = _NCHUNK // _NBUF
    lax.fori_loop(0, full, ring_body, 0)
    for g in range(full * _NBUF, _NCHUNK):
        b = g % _NBUF
        wait(g, r0s[b], r1s[b], s0s[b], s1s[b])
        compute(g, r0s[b], r1s[b])

    pltpu.sync_copy(ov, out.at[wid])


@jax.jit
def _run(x_disease, x_snorna, idx0, idx1):
    mesh = plsc.VectorSubcoreMesh(core_axis_name="c", subcore_axis_name="s")
    f = functools.partial(
        pl.kernel,
        mesh=mesh,
        out_type=jax.ShapeDtypeStruct((_NW, _PAD_W), jnp.float32),
        scratch_types=[
            pltpu.VMEM((_NCHUNK, _C), jnp.int32),
            pltpu.VMEM((_NCHUNK, _C), jnp.int32),
            [pltpu.VMEM((_C, D_FEAT // 2), jnp.int32)] * _NBUF,
            [pltpu.VMEM((_C, D_FEAT // 2), jnp.int32)] * _NBUF,
            pltpu.VMEM((_PAD_W,), jnp.float32),
            pltpu.VMEM((_C, 40), jnp.float32),
            [pltpu.SemaphoreType.DMA] * _NBUF,
            [pltpu.SemaphoreType.DMA] * _NBUF,
        ],
        compiler_params=pltpu.CompilerParams(needs_layout_passes=False,
                                             use_tc_tiling_on_sc=False),
    )(_sc_kernel)
    return f(x_disease, x_snorna, idx0, idx1)


def kernel(x_disease, x_snorna, edge_label_index):
    pad = ((0, 0), (0, _PAD_W - _PER_W))
    idx0 = jnp.pad(edge_label_index[0].reshape(_NW, _PER_W),
                   pad).reshape(_NW, _NCHUNK, _C)
    idx1 = jnp.pad(edge_label_index[1].reshape(_NW, _PER_W),
                   pad).reshape(_NW, _NCHUNK, _C)
    xd = lax.bitcast_convert_type(
        x_disease.astype(jnp.bfloat16).reshape(N_NODES, D_FEAT // 2, 2),
        jnp.int32)
    xs = lax.bitcast_convert_type(
        x_snorna.astype(jnp.bfloat16).reshape(N_NODES, D_FEAT // 2, 2),
        jnp.int32)
    return _run(xd, xs, idx0, idx1)[:, :_PER_W].reshape(N_EDGES)


# speedup vs baseline: 1.4161x; 1.3145x over previous
import functools

import jax
import jax.numpy as jnp
from jax import lax
from jax.experimental import pallas as pl
from jax.experimental.pallas import tpu as pltpu
from jax.experimental.pallas import tpu_sc as plsc

N_NODES = 10000
D_FEAT = 128
N_EDGES = 320000

_NC = 2
_NS = 16
_NW = _NC * _NS
_PER_W = N_EDGES // _NW
_C = 80
_NCHUNK = 125
_PAD_W = _NCHUNK * _C

_LANES = 16
_KVEC = D_FEAT // _LANES
_NBUF = 4


def _sc_kernel(xd, xs, idx0, idx1, out,
               i0all, i1all, r0s, r1s, ov, pv, s0s, s1s):
    wid = lax.axis_index("s") * _NC + lax.axis_index("c")
    pltpu.sync_copy(idx0.at[wid], i0all)
    pltpu.sync_copy(idx1.at[wid], i1all)

    lane = lax.iota(jnp.int32, _LANES)

    def issue(g, r0, r1, s0, s1):
        pltpu.async_copy(xd.at[i0all.at[g]], r0, s0)
        pltpu.async_copy(xs.at[i1all.at[g]], r1, s1)

    def wait(g, r0, r1, s0, s1):
        pltpu.make_async_copy(xd.at[i0all.at[g]], r0, s0).wait()
        pltpu.make_async_copy(xs.at[i1all.at[g]], r1, s1).wait()

    def compute(g, r0, r1):
        def p1_body(gr, gcarry):
            for t in range(2):
                base = gr * _LANES + t * 8
                ejs = [base + j for j in range(8)]
                blocks = []
                ss = ps = None

                def arith(k):
                    nonlocal ss, ps
                    aa, bb = blocks[k]
                    pp = [a * b for a, b in zip(aa, bb)]
                    if k == 0:
                        ss = pp
                    elif k == 1:
                        ps = pp
                    elif k == 2:
                        ss = [s + p for s, p in zip(ss, pp)]
                    else:
                        ps = [s + p for s, p in zip(ps, pp)]

                for k in range(_KVEC // 2):
                    aa = [plsc.bitcast(r0[e, pl.ds(k * _LANES, _LANES)],
                                       jnp.bfloat16) for e in ejs]
                    bb = [plsc.bitcast(r1[e, pl.ds(k * _LANES, _LANES)],
                                       jnp.bfloat16) for e in ejs]
                    blocks.append((aa, bb))
                    if k:
                        arith(k - 1)
                arith(_KVEC // 2 - 1)
                ss = [s + p for s, p in zip(ss, ps)]
                ups = [plsc.unpack(s, format=plsc.PackFormat.INTERLEAVED,
                                   preferred_element_type=jnp.float32)
                       for s in ss]
                res = [u0 + u1 for u0, u1 in ups]
                for j in range(8):
                    pv[base + j, pl.ds(0, _LANES)] = res[j]
            return gcarry

        lax.fori_loop(0, _C // _LANES, p1_body, 0)

        def p2_body(gr, gcarry):
            rows = gr * _LANES + lane
            cols = [plsc.load_gather(pv, [rows, jnp.full((_LANES,), c, jnp.int32)])
                    for c in range(_LANES)]
            for step in (8, 4, 2, 1):
                cols = [cols[2 * t] + cols[2 * t + 1] for t in range(step)]
            ov[pl.ds(g * _C + gr * _LANES, _LANES)] = cols[0]
            return gcarry

        lax.fori_loop(0, _C // _LANES, p2_body, 0)

    for b in range(_NBUF):
        issue(b, r0s[b], r1s[b], s0s[b], s1s[b])

    def ring_body(i, carry):
        for b in range(_NBUF):
            g = _NBUF * i + b
            wait(g, r0s[b], r1s[b], s0s[b], s1s[b])
            compute(g, r0s[b], r1s[b])

            @pl.when(g + _NBUF < _NCHUNK)
            def _():
                issue(g + _NBUF, r0s[b], r1s[b], s0s[b], s1s[b])

        return carry

    full = _NCHUNK // _NBUF
    lax.fori_loop(0, full, ring_body, 0)
    for g in range(full * _NBUF, _NCHUNK):
        b = g % _NBUF
        wait(g, r0s[b], r1s[b], s0s[b], s1s[b])
        compute(g, r0s[b], r1s[b])

    pltpu.sync_copy(ov, out.at[wid])


@jax.jit
def _run(x_disease, x_snorna, idx0, idx1):
    mesh = plsc.VectorSubcoreMesh(core_axis_name="c", subcore_axis_name="s")
    f = functools.partial(
        pl.kernel,
        mesh=mesh,
        out_type=jax.ShapeDtypeStruct((_NW, _PAD_W), jnp.float32),
        scratch_types=[
            pltpu.VMEM((_NCHUNK, _C), jnp.int32),
            pltpu.VMEM((_NCHUNK, _C), jnp.int32),
            [pltpu.VMEM((_C, D_FEAT // 2), jnp.int32)] * _NBUF,
            [pltpu.VMEM((_C, D_FEAT // 2), jnp.int32)] * _NBUF,
            pltpu.VMEM((_PAD_W,), jnp.float32),
            pltpu.VMEM((_C, 40), jnp.float32),
            [pltpu.SemaphoreType.DMA] * _NBUF,
            [pltpu.SemaphoreType.DMA] * _NBUF,
        ],
        compiler_params=pltpu.CompilerParams(needs_layout_passes=False,
                                             use_tc_tiling_on_sc=False),
    )(_sc_kernel)
    return f(x_disease, x_snorna, idx0, idx1)


def kernel(x_disease, x_snorna, edge_label_index):
    pad = ((0, 0), (0, _PAD_W - _PER_W))
    idx0 = jnp.pad(edge_label_index[0].reshape(_NW, _PER_W),
                   pad).reshape(_NW, _NCHUNK, _C)
    idx1 = jnp.pad(edge_label_index[1].reshape(_NW, _PER_W),
                   pad).reshape(_NW, _NCHUNK, _C)
    xd = lax.bitcast_convert_type(
        x_disease.astype(jnp.bfloat16).reshape(N_NODES, D_FEAT // 2, 2),
        jnp.int32)
    xs = lax.bitcast_convert_type(
        x_snorna.astype(jnp.bfloat16).reshape(N_NODES, D_FEAT // 2, 2),
        jnp.int32)
    return _run(xd, xs, idx0, idx1)[:, :_PER_W].reshape(N_EDGES)
